# Initial kernel scaffold; baseline (speedup 1.0000x reference)
import jax
import jax.numpy as jnp
from jax.experimental import pallas as pl


def kernel(x, seg, token_table, pos_table, seg_table, gamma, beta):
    raise NotImplementedError("write your pallas kernel here")


# speedup vs baseline: 4.2256x; 4.2256x over previous
import functools

import jax
import jax.numpy as jnp
from jax import lax
from jax.experimental import pallas as pl
from jax.experimental.pallas import tpu as pltpu
from jax.experimental.pallas import tpu_sc as plsc

LANES = 16
NC = 2
NS = 16
NW = NC * NS

D = 64
ND = D // LANES

_GDN = lax.GatherDimensionNumbers(
    offset_dims=(), collapsed_slice_dims=(0,), start_index_map=(0,))


def _lane_bcast(v, lane):
    idx = jnp.full((LANES, 1), lane, dtype=jnp.int32)
    return lax.gather(v, idx, _GDN, (1,),
                      mode=lax.GatherScatterMode.PROMISE_IN_BOUNDS)


def _rsqrt_vec(x):
    i = lax.bitcast_convert_type(x, jnp.int32)
    i = jnp.int32(0x5F3759DF) - lax.shift_right_arithmetic(i, jnp.int32(1))
    y = lax.bitcast_convert_type(i, jnp.float32)
    for _ in range(2):
        y = y * (1.5 - 0.5 * x * y * y)
    return y


def _make_kernel(B, L, V):
    assert B % NW == 0
    nseq = B // NW
    s0, s1 = 104, L - 104
    ngrp = L // LANES
    tail = L - ngrp * LANES

    mesh = plsc.VectorSubcoreMesh(core_axis_name="c", subcore_axis_name="s")

    @functools.partial(
        pl.kernel,
        out_type=jax.ShapeDtypeStruct((B, L, D), jnp.float32),
        mesh=mesh,
        compiler_params=pltpu.CompilerParams(
            needs_layout_passes=False, use_tc_tiling_on_sc=False),
        scratch_types=[
            pltpu.VMEM((L,), jnp.int32),
            pltpu.VMEM((L + 16,), jnp.int32),
            pltpu.VMEM((L, D), jnp.float32),
            pltpu.VMEM((L, D), jnp.float32),
            pltpu.VMEM((2, D), jnp.float32),
            pltpu.VMEM((D,), jnp.float32),
            pltpu.VMEM((D,), jnp.float32),
            pltpu.VMEM((D,), jnp.float32),
            pltpu.SemaphoreType.DMA,
        ],
    )
    def k(x_hbm, seg_hbm, tok_hbm, pos_hbm, segt_hbm, gam_hbm, bet_hbm,
          out_hbm, idx_v, segv, rows, posb, segt_v, dd_v, gm_v, bt_v, sem):
        wid = lax.axis_index("s") * NC + lax.axis_index("c")
        seq0 = wid * nseq

        pltpu.sync_copy(pos_hbm.at[pl.ds(0, L)], posb)
        pltpu.sync_copy(segt_hbm, segt_v)
        pltpu.sync_copy(gam_hbm, gm_v)
        pltpu.sync_copy(bet_hbm, bt_v)

        sl = [pl.ds(kk * LANES, LANES) for kk in range(ND)]

        def _fold_seg0(r, carry):
            for kk in range(ND):
                posb[r, sl[kk]] = posb[r, sl[kk]] + segt_v[0, sl[kk]]
            return carry
        lax.fori_loop(0, L, _fold_seg0, 0)
        for kk in range(ND):
            dd_v[sl[kk]] = segt_v[1, sl[kk]] - segt_v[0, sl[kk]]

        def _token(t, i, sf, dd, gm, bt):
            tok = [rows[t, sl[kk]] for kk in range(ND)]
            pos = [posb[t, sl[kk]] for kk in range(ND)]
            sb = _lane_bcast(sf, i)
            e = [tok[kk] + pos[kk] + sb * dd[kk] for kk in range(ND)]
            tot = (e[0] + e[1]) + (e[2] + e[3])
            mean = _lane_bcast(plsc.cumsum(tot), LANES - 1) * (1.0 / D)
            c = [e[kk] - mean for kk in range(ND)]
            sq = (c[0] * c[0] + c[1] * c[1]) + (c[2] * c[2] + c[3] * c[3])
            var = _lane_bcast(plsc.cumsum(sq), LANES - 1) * (1.0 / D)
            rv = _rsqrt_vec(var + 1e-5)
            for kk in range(ND):
                rows[t, sl[kk]] = c[kk] * (rv * gm[kk]) + bt[kk]

        def _seq_body(s, carry):
            gseq = seq0 + s
            pltpu.sync_copy(x_hbm.at[gseq], idx_v)
            pltpu.sync_copy(seg_hbm.at[gseq], segv.at[pl.ds(0, L)])
            c1 = pltpu.async_copy(tok_hbm.at[idx_v.at[pl.ds(0, s0)]],
                                  rows.at[pl.ds(0, s0), :], sem)
            c2 = pltpu.async_copy(tok_hbm.at[idx_v.at[pl.ds(s0, s1)]],
                                  rows.at[pl.ds(s0, s1), :], sem)
            c1.wait()
            c2.wait()

            dd = [dd_v[sl[kk]] for kk in range(ND)]
            gm = [gm_v[sl[kk]] for kk in range(ND)]
            bt = [bt_v[sl[kk]] for kk in range(ND)]

            def _grp(g, carry2):
                base = g * LANES
                sf = segv[pl.ds(base, LANES)].astype(jnp.float32)
                for i in range(LANES):
                    _token(base + i, i, sf, dd, gm, bt)
                return carry2
            lax.fori_loop(0, ngrp, _grp, 0)

            base = ngrp * LANES
            sf = segv[pl.ds(base, LANES)].astype(jnp.float32)
            for i in range(tail):
                _token(base + i, i, sf, dd, gm, bt)

            pltpu.sync_copy(rows, out_hbm.at[gseq])
            return carry
        lax.fori_loop(0, nseq, _seq_body, 0)

    return k


def kernel(x, seg, token_table, pos_table, seg_table, gamma, beta):
    B, L = x.shape
    V, d = token_table.shape
    k = _make_kernel(B, L, V)
    return k(x.astype(jnp.int32), seg.astype(jnp.int32),
             token_table, pos_table, seg_table, gamma, beta)


# speedup vs baseline: 5.4696x; 1.2944x over previous
import functools

import jax
import jax.numpy as jnp
from jax import lax
from jax.experimental import pallas as pl
from jax.experimental.pallas import tpu as pltpu
from jax.experimental.pallas import tpu_sc as plsc

LANES = 16
NC = 2
NS = 16
NW = NC * NS

D = 64
ND = D // LANES
IB = 2
NBUF = 3

_GDN = lax.GatherDimensionNumbers(
    offset_dims=(), collapsed_slice_dims=(0,), start_index_map=(0,))


def _lane_bcast(v, lane):
    idx = jnp.full((LANES, 1), lane, dtype=jnp.int32)
    return lax.gather(v, idx, _GDN, (1,),
                      mode=lax.GatherScatterMode.PROMISE_IN_BOUNDS)


def _rsqrt_vec(x):
    i = lax.bitcast_convert_type(x, jnp.int32)
    i = jnp.int32(0x5F3759DF) - lax.shift_right_arithmetic(i, jnp.int32(1))
    y = lax.bitcast_convert_type(i, jnp.float32)
    for _ in range(2):
        y = y * (1.5 - 0.5 * x * y * y)
    return y


def _make_kernel(B, L, V):
    assert B % (NW * IB) == 0
    nblk = B // (NW * IB)
    s0, s1 = 104, L - 104
    ngrp = L // LANES
    tail = L - ngrp * LANES

    mesh = plsc.VectorSubcoreMesh(core_axis_name="c", subcore_axis_name="s")

    scratch = (
        [pltpu.VMEM((IB, L, D), jnp.float32) for _ in range(NBUF)]
        + [pltpu.VMEM((IB, L), jnp.int32) for _ in range(NBUF)]
        + [pltpu.VMEM((IB, L), jnp.int32) for _ in range(NBUF)]
        + [
            pltpu.VMEM((L, D), jnp.float32),
            pltpu.VMEM((2, D), jnp.float32),
            pltpu.VMEM((D,), jnp.float32),
            pltpu.VMEM((D,), jnp.float32),
            pltpu.VMEM((D,), jnp.float32),
        ]
        + [pltpu.SemaphoreType.DMA for _ in range(3 * NBUF)]
    )

    @functools.partial(
        pl.kernel,
        out_type=jax.ShapeDtypeStruct((B, L, D), jnp.float32),
        mesh=mesh,
        compiler_params=pltpu.CompilerParams(
            needs_layout_passes=False, use_tc_tiling_on_sc=False),
        scratch_types=scratch,
    )
    def k(x_hbm, seg_hbm, tok_hbm, pos_hbm, segt_hbm, gam_hbm, bet_hbm,
          out_hbm, *refs):
        rows = refs[0:NBUF]
        idxb = refs[NBUF:2 * NBUF]
        segb = refs[2 * NBUF:3 * NBUF]
        posb, segt_v, dd_v, gm_v, bt_v = refs[3 * NBUF:3 * NBUF + 5]
        sems = refs[3 * NBUF + 5:]
        sem_g = sems[0:NBUF]
        sem_i = sems[NBUF:2 * NBUF]
        sem_o = sems[2 * NBUF:]

        wid = lax.axis_index("s") * NC + lax.axis_index("c")
        seq00 = wid * (nblk * IB)

        pltpu.sync_copy(pos_hbm.at[pl.ds(0, L)], posb)
        pltpu.sync_copy(segt_hbm, segt_v)
        pltpu.sync_copy(gam_hbm, gm_v)
        pltpu.sync_copy(bet_hbm, bt_v)

        sl = [pl.ds(kk * LANES, LANES) for kk in range(ND)]

        def _fold_seg0(r, carry):
            for kk in range(ND):
                posb[r, sl[kk]] = posb[r, sl[kk]] + segt_v[0, sl[kk]]
            return carry
        lax.fori_loop(0, L, _fold_seg0, 0)
        for kk in range(ND):
            dd_v[sl[kk]] = segt_v[1, sl[kk]] - segt_v[0, sl[kk]]

        def seqbase(m):
            return seq00 + m * IB

        def stage(m, b, issue):
            src_x = x_hbm.at[pl.ds(seqbase(m), IB)]
            src_s = seg_hbm.at[pl.ds(seqbase(m), IB)]
            if issue:
                pltpu.async_copy(src_x, idxb[b], sem_i[b])
                pltpu.async_copy(src_s, segb[b], sem_i[b])
            else:
                pltpu.make_async_copy(src_x, idxb[b], sem_i[b]).wait()
                pltpu.make_async_copy(src_s, segb[b], sem_i[b]).wait()

        def gathers(b, issue):
            for q in range(IB):
                for (off, n) in ((0, s0), (s0, s1)):
                    src = tok_hbm.at[idxb[b].at[q, pl.ds(off, n)]]
                    dst = rows[b].at[q, pl.ds(off, n), :]
                    if issue:
                        pltpu.async_copy(src, dst, sem_g[b])
                    else:
                        pltpu.make_async_copy(src, dst, sem_g[b]).wait()

        def out_dma(m, b, issue):
            dst = out_hbm.at[pl.ds(seqbase(m), IB)]
            if issue:
                pltpu.async_copy(rows[b], dst, sem_o[b])
            else:
                pltpu.make_async_copy(rows[b], dst, sem_o[b]).wait()

        def _token(rv, q, t, i, sf, dd, gm, bt):
            tok = [rv[q, t, sl[kk]] for kk in range(ND)]
            pos = [posb[t, sl[kk]] for kk in range(ND)]
            sb = _lane_bcast(sf, i)
            e = [tok[kk] + pos[kk] + sb * dd[kk] for kk in range(ND)]
            tot = (e[0] + e[1]) + (e[2] + e[3])
            mean = _lane_bcast(plsc.cumsum(tot), LANES - 1) * (1.0 / D)
            c = [e[kk] - mean for kk in range(ND)]
            sq = (c[0] * c[0] + c[1] * c[1]) + (c[2] * c[2] + c[3] * c[3])
            var = _lane_bcast(plsc.cumsum(sq), LANES - 1) * (1.0 / D)
            rv_ = _rsqrt_vec(var + 1e-5)
            for kk in range(ND):
                rv[q, t, sl[kk]] = c[kk] * (rv_ * gm[kk]) + bt[kk]

        def compute(b):
            dd = [dd_v[sl[kk]] for kk in range(ND)]
            gm = [gm_v[sl[kk]] for kk in range(ND)]
            bt = [bt_v[sl[kk]] for kk in range(ND)]
            for q in range(IB):
                def _grp(g, carry, q=q):
                    base = g * LANES
                    sf = segb[b][q, pl.ds(base, LANES)].astype(jnp.float32)
                    for i in range(LANES):
                        _token(rows[b], q, base + i, i, sf, dd, gm, bt)
                    return carry
                lax.fori_loop(0, ngrp, _grp, 0)
                base = L - LANES
                sf = segb[b][q, pl.ds(base, LANES)].astype(jnp.float32)
                for i in range(LANES - tail, LANES):
                    _token(rows[b], q, base + i, i, sf, dd, gm, bt)

        pltpu.sync_copy(x_hbm.at[pl.ds(seqbase(0), IB)], idxb[0])
        pltpu.sync_copy(seg_hbm.at[pl.ds(seqbase(0), IB)], segb[0])
        stage(1, 1, True)
        gathers(0, True)

        def _iter(p, carry):
            for kk3 in range(NBUF):
                m = p * NBUF + kk3
                b = kk3
                bn = (kk3 + 1) % NBUF
                bs = (kk3 + 2) % NBUF
                gathers(b, False)
                stage(m + 1, bn, False)
                if kk3 == 2:
                    out_dma(m - 2, bn, False)
                else:
                    @pl.when(p >= 1)
                    def _w():
                        out_dma(m - 2, bn, False)
                gathers(bn, True)
                if kk3 == 2:
                    @pl.when(p <= (nblk - 5) // NBUF)
                    def _s():
                        stage(m + 2, bs, True)
                else:
                    stage(m + 2, bs, True)
                compute(b)
                out_dma(m, b, True)
            return carry
        lax.fori_loop(0, (nblk - 1) // NBUF, _iter, 0)

        mlast = nblk - 1
        gathers(0, False)
        compute(0)
        out_dma(mlast, 0, True)
        out_dma(mlast - 2, 1, False)
        out_dma(mlast - 1, 2, False)
        out_dma(mlast, 0, False)

    return k


def kernel(x, seg, token_table, pos_table, seg_table, gamma, beta):
    B, L = x.shape
    V, d = token_table.shape
    k = _make_kernel(B, L, V)
    return k(x.astype(jnp.int32), seg.astype(jnp.int32),
             token_table, pos_table, seg_table, gamma, beta)
